# baseline (speedup 1.0000x reference)
import jax
import jax.numpy as jnp
from jax import lax
from jax.experimental import pallas as pl
from jax.experimental.pallas import tpu as pltpu
from jax.experimental.pallas import tpu_sc as plsc

N = 10000
E = 320000
H = 128
HH = H // 2
IN = 128
WL_DIM = 16
NUM_WL = 1000

NC = 2
NS = 16
NW = NC * NS

EPW = E // NW
CHUNK = 80
NCHUNK = EPW // CHUNK
RING = 5

ROWS_BIG = 640
ROWS_LAST = N - 15 * ROWS_BIG
ZROWS = 80

WL_WORKERS = 25
WL_PER_W = N // WL_WORKERS
WL_CHUNK = 80


def _sc_body(src_hbm, dst_hbm, ew_hbm, wA_hbm, wB_hbm, wlidx_hbm, wltab_hbm,
             partA_hbm, partB_hbm, wl_out_hbm,
             acc, table,
             rows0, rows1, rows2, rows3, rows4,
             sv0, sv1, sv2, sv3, sv4,
             dv0, dv1, dv2, dv3, dv4,
             wv0, wv1, wv2, wv3, wv4,
             wl_idx_v,
             gsem0, gsem1, gsem2, gsem3, gsem4,
             ssem0, ssem1, ssem2, ssem3, ssem4,
             isem0, isem1, isem2, isem3, isem4, wlsem):
  c = lax.axis_index("c")
  s = lax.axis_index("s")
  w = c * NS + s
  rows = [rows0, rows1, rows2, rows3, rows4]
  srcv = [sv0, sv1, sv2, sv3, sv4]
  dstv = [dv0, dv1, dv2, dv3, dv4]
  wv = [wv0, wv1, wv2, wv3, wv4]
  gsems = [gsem0, gsem1, gsem2, gsem3, gsem4]
  ssems = [ssem0, ssem1, ssem2, ssem3, ssem4]
  isems = [isem0, isem1, isem2, isem3, isem4]

  zeros16 = jnp.zeros((16,), jnp.float32)

  def fire_trio(jj, slot):
    pltpu.async_copy(src_hbm.at[w, jj], srcv[slot], isems[slot])
    pltpu.async_copy(dst_hbm.at[w, jj], dstv[slot], isems[slot])
    pltpu.async_copy(ew_hbm.at[w, jj], wv[slot], isems[slot])

  def wait_trio(slot):
    pltpu.make_async_copy(src_hbm.at[0, 0], srcv[slot], isems[slot]).wait()
    pltpu.make_async_copy(dst_hbm.at[0, 0], dstv[slot], isems[slot]).wait()
    pltpu.make_async_copy(ew_hbm.at[0, 0], wv[slot], isems[slot]).wait()

  def drain_rows(sem, slot):
    pltpu.make_async_copy(wA_hbm.at[pl.ds(0, CHUNK)], rows[slot], sem).wait()

  @pl.when(w < WL_WORKERS)
  def _():
    def wl_chunk(j, _):
      off = w * WL_PER_W + j * WL_CHUNK
      pltpu.sync_copy(wlidx_hbm.at[pl.ds(off, WL_CHUNK)], wl_idx_v)
      pltpu.async_copy(wltab_hbm.at[wl_idx_v], rows0, wlsem).wait()
      pltpu.sync_copy(rows0, wl_out_hbm.at[pl.ds(off, WL_CHUNK)])
      return 0

    lax.fori_loop(0, WL_PER_W // WL_CHUNK, wl_chunk, 0)

  start = s * ROWS_BIG
  nz = jnp.where(s == NS - 1, ROWS_LAST // ZROWS, ROWS_BIG // ZROWS)

  def run_phase(wedge_hbm, part_hbm):
    @pl.when(s < NS - 1)
    def _():
      pltpu.sync_copy(wedge_hbm.at[pl.ds(s * ROWS_BIG, ROWS_BIG)],
                      table.at[pl.ds(s * ROWS_BIG, ROWS_BIG)])

    @pl.when(s == NS - 1)
    def _():
      pltpu.sync_copy(wedge_hbm.at[pl.ds((NS - 1) * ROWS_BIG, ROWS_LAST)],
                      table.at[pl.ds((NS - 1) * ROWS_BIG, ROWS_LAST)])

    def zero_row(r, _):
      for hh in range(HH // 16):
        rows0[r, pl.ds(hh * 16, 16)] = zeros16
      return 0

    lax.fori_loop(0, ZROWS, zero_row, 0)

    def zero_chunk(j, _):
      pltpu.sync_copy(rows0, acc.at[pl.ds(start + j * ZROWS, ZROWS)])
      return 0

    lax.fori_loop(0, nz, zero_chunk, 0)
    plsc.subcore_barrier()

    fire_trio(0, 0)
    fire_trio(1, 1)
    fire_trio(2, 2)
    wait_trio(0)
    pltpu.async_copy(table.at[srcv[0]], rows[0], gsems[0])
    wait_trio(1)
    pltpu.async_copy(table.at[srcv[1]], rows[1], gsems[1])

    def quint(k, _):
      for b in range(RING):
        j = RING * k + b
        drain_rows(gsems[b], b)

        def scale_group(g, _, b=b):
          w16 = wv[b][pl.ds(g * 16, 16)]
          for l in range(16):
            wspl = w16.at[jnp.full((16,), l, jnp.int32)].get(
                mode="promise_in_bounds")
            e = g * 16 + l
            for hh in range(HH // 16):
              rows[b][e, pl.ds(hh * 16, 16)] = (
                  rows[b][e, pl.ds(hh * 16, 16)] * wspl)
          return 0

        lax.fori_loop(0, CHUNK // 16, scale_group, 0)
        pltpu.async_copy(rows[b], acc.at[dstv[b]], ssems[b], add=True)

        j3 = j + 3
        b3 = (b + 3) % RING

        @pl.when(j3 < NCHUNK)
        def _(j3=j3, b3=b3):
          @pl.when(j3 >= RING)
          def _():
            drain_rows(ssems[b3], b3)

          fire_trio(j3, b3)

        j2 = j + 2
        b2 = (b + 2) % RING

        @pl.when(j2 < NCHUNK)
        def _(j2=j2, b2=b2):
          wait_trio(b2)
          pltpu.async_copy(table.at[srcv[b2]], rows[b2], gsems[b2])

      return 0

    lax.fori_loop(0, NCHUNK // RING, quint, 0)
    for b in range(RING):
      drain_rows(ssems[b], b)

    plsc.subcore_barrier()

    @pl.when(s < NS - 1)
    def _():
      pltpu.sync_copy(acc.at[pl.ds(s * ROWS_BIG, ROWS_BIG)],
                      part_hbm.at[c, pl.ds(s * ROWS_BIG, ROWS_BIG)])

    @pl.when(s == NS - 1)
    def _():
      pltpu.sync_copy(acc.at[pl.ds((NS - 1) * ROWS_BIG, ROWS_LAST)],
                      part_hbm.at[c, pl.ds((NS - 1) * ROWS_BIG, ROWS_LAST)])

  run_phase(wA_hbm, partA_hbm)
  run_phase(wB_hbm, partB_hbm)


def _sc_spmm(src3, dst3, ew3, W_A, W_B, wl_indices, wl_table_pad):
  mesh = plsc.VectorSubcoreMesh(core_axis_name="c", subcore_axis_name="s",
                                num_cores=NC, num_subcores=NS)
  f = pl.kernel(
      _sc_body,
      out_type=(
          jax.ShapeDtypeStruct((NC, N, HH), jnp.float32),
          jax.ShapeDtypeStruct((NC, N, HH), jnp.float32),
          jax.ShapeDtypeStruct((N, HH), jnp.float32),
      ),
      mesh=mesh,
      scratch_types=(
          [
              pltpu.VMEM_SHARED((N, HH), jnp.float32),
              pltpu.VMEM_SHARED((N, HH), jnp.float32),
          ]
          + [pltpu.VMEM((CHUNK, HH), jnp.float32) for _ in range(RING)]
          + [pltpu.VMEM((CHUNK,), jnp.int32) for _ in range(RING)]
          + [pltpu.VMEM((CHUNK,), jnp.int32) for _ in range(RING)]
          + [pltpu.VMEM((CHUNK,), jnp.float32) for _ in range(RING)]
          + [pltpu.VMEM((WL_CHUNK,), jnp.int32)]
          + [pltpu.SemaphoreType.DMA for _ in range(3 * RING + 1)]
      ),
      compiler_params=pltpu.CompilerParams(use_tc_tiling_on_sc=False),
  )
  return f(src3, dst3, ew3, W_A, W_B, wl_indices, wl_table_pad)


BN = 2000


def _tc_body(pA, pB, x, wl_e, b_edge, c1W, c1b, nW0, nW1, nb, c2W, c2b,
             fW, fb, out):
  f32 = jnp.float32
  s1A = pA[0] + pA[1] + b_edge[0, :HH]
  s1B = pB[0] + pB[1] + b_edge[0, HH:]
  ri = lax.broadcasted_iota(jnp.int32, (HH, H), 0)
  ci = lax.broadcasted_iota(jnp.int32, (HH, H), 1)
  m1A = c1W[:HH, :] + (ci == ri).astype(f32)
  m1B = c1W[HH:, :] + (ci == ri + HH).astype(f32)
  t = (jnp.dot(s1A, m1A, preferred_element_type=f32)
       + jnp.dot(s1B, m1B, preferred_element_type=f32) + c1b[:])
  xn = (jnp.dot(x[:], nW0[:], preferred_element_type=f32)
        + jnp.dot(wl_e[:, :WL_DIM], nW1[:], preferred_element_type=f32)
        + nb[:])
  t = t + xn + jnp.dot(xn, c2W[:], preferred_element_type=f32) + c2b[:]
  out[:] = jnp.dot(jnp.maximum(t, 0.0), fW[:], preferred_element_type=f32) + fb[:]


def _tc_dense(pA, pB, x, wl_e, b_edge, c1W, c1b, nW0, nW1, nb, c2W, c2b,
              fW, fb):
  grid = (N // BN,)
  in_specs = [
      pl.BlockSpec((NC, BN, HH), lambda i: (0, i, 0)),
      pl.BlockSpec((NC, BN, HH), lambda i: (0, i, 0)),
      pl.BlockSpec((BN, IN), lambda i: (i, 0)),
      pl.BlockSpec((BN, HH), lambda i: (i, 0)),
      pl.BlockSpec((1, H), lambda i: (0, 0)),
      pl.BlockSpec((H, H), lambda i: (0, 0)),
      pl.BlockSpec((1, H), lambda i: (0, 0)),
      pl.BlockSpec((IN, H), lambda i: (0, 0)),
      pl.BlockSpec((WL_DIM, H), lambda i: (0, 0)),
      pl.BlockSpec((1, H), lambda i: (0, 0)),
      pl.BlockSpec((H, H), lambda i: (0, 0)),
      pl.BlockSpec((1, H), lambda i: (0, 0)),
      pl.BlockSpec((H, H), lambda i: (0, 0)),
      pl.BlockSpec((1, H), lambda i: (0, 0)),
  ]
  return pl.pallas_call(
      _tc_body,
      grid=grid,
      in_specs=in_specs,
      out_specs=pl.BlockSpec((BN, H), lambda i: (i, 0)),
      out_shape=jax.ShapeDtypeStruct((N, H), jnp.float32),
  )(pA, pB, x, wl_e, b_edge, c1W, c1b, nW0, nW1, nb, c2W, c2b, fW, fb)


def kernel(wl_indices, x, edge_index, edge_weight, W_edge, b_edge, wl_table,
           node_W, node_b, cat1_W, cat1_b, cat2_W, cat2_b, final_W, final_b):
  src3 = edge_index[0].reshape(NW, NCHUNK, CHUNK)
  dst3 = edge_index[1].reshape(NW, NCHUNK, CHUNK)
  ew3 = edge_weight.reshape(NW, NCHUNK, CHUNK)
  W_A = W_edge[:, :HH]
  W_B = W_edge[:, HH:]
  wl_pad = jnp.pad(wl_table, ((0, 0), (0, HH - WL_DIM)))
  pA, pB, wl_e = _sc_spmm(src3, dst3, ew3, W_A, W_B, wl_indices, wl_pad)
  nW0 = node_W[:IN]
  nW1 = node_W[IN:]
  r = lambda v: v.reshape(1, -1)
  return _tc_dense(pA, pB, x, wl_e, r(b_edge), cat1_W, r(cat1_b), nW0, nW1,
                   r(node_b), cat2_W, r(cat2_b), final_W, r(final_b))


# speedup vs baseline: 1.2264x; 1.2264x over previous
import jax
import jax.numpy as jnp
from jax import lax
from jax.experimental import pallas as pl
from jax.experimental.pallas import tpu as pltpu
from jax.experimental.pallas import tpu_sc as plsc

N = 10000
E = 320000
H = 128
HH = H // 2
IN = 128
WL_DIM = 16
NUM_WL = 1000

NC = 2
NS = 16
NW = NC * NS

EPW = E // NW
CHUNK = 80
NCHUNK = EPW // CHUNK
RING = 5

ROWS_BIG = 640
ROWS_LAST = N - 15 * ROWS_BIG
ZROWS = 80

WL_WORKERS = 25
WL_PER_W = N // WL_WORKERS
WL_CHUNK = 80


def _sc_body(src_hbm, dst_hbm, ew_hbm, wA_hbm, wB_hbm, wlidx_hbm, wltab_hbm,
             partA_hbm, partB_hbm, wl_out_hbm,
             acc, table,
             rows0, rows1, rows2, rows3, rows4,
             fr0, fr1, fr2, fr3, fr4,
             sv0, sv1, sv2, sv3, sv4,
             dv0, dv1, dv2, dv3, dv4,
             wv0, wv1, wv2, wv3, wv4,
             wl_idx_v,
             gsem0, gsem1, gsem2, gsem3, gsem4,
             ssem0, ssem1, ssem2, ssem3, ssem4,
             isem0, isem1, isem2, isem3, isem4, wlsem):
  c = lax.axis_index("c")
  s = lax.axis_index("s")
  w = c * NS + s
  rows = [rows0, rows1, rows2, rows3, rows4]
  frs = [fr0, fr1, fr2, fr3, fr4]
  srcv = [sv0, sv1, sv2, sv3, sv4]
  dstv = [dv0, dv1, dv2, dv3, dv4]
  wv = [wv0, wv1, wv2, wv3, wv4]
  gsems = [gsem0, gsem1, gsem2, gsem3, gsem4]
  ssems = [ssem0, ssem1, ssem2, ssem3, ssem4]
  isems = [isem0, isem1, isem2, isem3, isem4]

  zeros16 = jnp.zeros((16,), jnp.float32)

  def fire_trio(jj, slot):
    pltpu.async_copy(src_hbm.at[w, jj], srcv[slot], isems[slot])
    pltpu.async_copy(dst_hbm.at[w, jj], dstv[slot], isems[slot])
    pltpu.async_copy(ew_hbm.at[w, jj], wv[slot], isems[slot])

  def wait_trio(slot):
    pltpu.make_async_copy(src_hbm.at[0, 0], srcv[slot], isems[slot]).wait()
    pltpu.make_async_copy(dst_hbm.at[0, 0], dstv[slot], isems[slot]).wait()
    pltpu.make_async_copy(ew_hbm.at[0, 0], wv[slot], isems[slot]).wait()

  def drain_gather(sem, slot):
    pltpu.make_async_copy(wA_hbm.at[pl.ds(0, CHUNK)], rows[slot], sem).wait()

  def drain_scatter(sem, slot):
    pltpu.make_async_copy(partA_hbm.at[0, pl.ds(0, CHUNK)], frs[slot],
                          sem).wait()

  @pl.when(w < WL_WORKERS)
  def _():
    def wl_chunk(j, _):
      off = w * WL_PER_W + j * WL_CHUNK
      pltpu.sync_copy(wlidx_hbm.at[pl.ds(off, WL_CHUNK)], wl_idx_v)
      pltpu.async_copy(wltab_hbm.at[wl_idx_v], fr0, wlsem).wait()
      pltpu.sync_copy(fr0, wl_out_hbm.at[pl.ds(off, WL_CHUNK)])
      return 0

    lax.fori_loop(0, WL_PER_W // WL_CHUNK, wl_chunk, 0)

  start = s * ROWS_BIG
  nz = jnp.where(s == NS - 1, ROWS_LAST // ZROWS, ROWS_BIG // ZROWS)

  def run_phase(wedge_hbm, part_hbm):
    @pl.when(s < NS - 1)
    def _():
      pltpu.sync_copy(wedge_hbm.at[pl.ds(s * ROWS_BIG, ROWS_BIG)],
                      table.at[pl.ds(s * ROWS_BIG, ROWS_BIG)])

    @pl.when(s == NS - 1)
    def _():
      pltpu.sync_copy(wedge_hbm.at[pl.ds((NS - 1) * ROWS_BIG, ROWS_LAST)],
                      table.at[pl.ds((NS - 1) * ROWS_BIG, ROWS_LAST)])

    def zero_row(r, _):
      for hh in range(HH // 16):
        fr0[r, pl.ds(hh * 16, 16)] = zeros16
      return 0

    lax.fori_loop(0, ZROWS, zero_row, 0)

    def zero_chunk(j, _):
      pltpu.sync_copy(fr0, acc.at[pl.ds(start + j * ZROWS, ZROWS)])
      return 0

    lax.fori_loop(0, nz, zero_chunk, 0)
    plsc.subcore_barrier()

    fire_trio(0, 0)
    fire_trio(1, 1)
    fire_trio(2, 2)
    wait_trio(0)
    pltpu.async_copy(table.at[srcv[0]], rows[0], gsems[0])
    wait_trio(1)
    pltpu.async_copy(table.at[srcv[1]], rows[1], gsems[1])

    def quint(k, _):
      for b in range(RING):
        j = RING * k + b
        drain_gather(gsems[b], b)

        def scale_group(g, _, b=b):
          w16 = wv[b][pl.ds(g * 16, 16)]
          for l in range(16):
            wspl = w16.at[jnp.full((16,), l, jnp.int32)].get(
                mode="promise_in_bounds")
            e = g * 16 + l
            for gg in range(HH // 32):
              v = rows[b][e, pl.ds(gg * 32, 32)]
              va, vb = plsc.unpack(v, format=plsc.PackFormat.INTERLEAVED)
              frs[b][e, pl.ds(gg * 32, 16)] = va * wspl
              frs[b][e, pl.ds(gg * 32 + 16, 16)] = vb * wspl
          return 0

        lax.fori_loop(0, CHUNK // 16, scale_group, 0)
        pltpu.async_copy(frs[b], acc.at[dstv[b]], ssems[b], add=True)

        j3 = j + 3
        b3 = (b + 3) % RING

        @pl.when(j3 < NCHUNK)
        def _(j3=j3, b3=b3):
          @pl.when(j3 >= RING)
          def _():
            drain_scatter(ssems[b3], b3)

          fire_trio(j3, b3)

        j2 = j + 2
        b2 = (b + 2) % RING

        @pl.when(j2 < NCHUNK)
        def _(j2=j2, b2=b2):
          wait_trio(b2)
          pltpu.async_copy(table.at[srcv[b2]], rows[b2], gsems[b2])

      return 0

    lax.fori_loop(0, NCHUNK // RING, quint, 0)
    for b in range(RING):
      drain_scatter(ssems[b], b)

    plsc.subcore_barrier()

    @pl.when(s < NS - 1)
    def _():
      pltpu.sync_copy(acc.at[pl.ds(s * ROWS_BIG, ROWS_BIG)],
                      part_hbm.at[c, pl.ds(s * ROWS_BIG, ROWS_BIG)])

    @pl.when(s == NS - 1)
    def _():
      pltpu.sync_copy(acc.at[pl.ds((NS - 1) * ROWS_BIG, ROWS_LAST)],
                      part_hbm.at[c, pl.ds((NS - 1) * ROWS_BIG, ROWS_LAST)])

  run_phase(wA_hbm, partA_hbm)
  run_phase(wB_hbm, partB_hbm)


def _sc_spmm(src3, dst3, ew3, W_A, W_B, wl_indices, wl_table_pad):
  mesh = plsc.VectorSubcoreMesh(core_axis_name="c", subcore_axis_name="s",
                                num_cores=NC, num_subcores=NS)
  f = pl.kernel(
      _sc_body,
      out_type=(
          jax.ShapeDtypeStruct((NC, N, HH), jnp.float32),
          jax.ShapeDtypeStruct((NC, N, HH), jnp.float32),
          jax.ShapeDtypeStruct((N, HH), jnp.float32),
      ),
      mesh=mesh,
      scratch_types=(
          [
              pltpu.VMEM_SHARED((N, HH), jnp.float32),
              pltpu.VMEM_SHARED((N, HH), jnp.bfloat16),
          ]
          + [pltpu.VMEM((CHUNK, HH), jnp.bfloat16) for _ in range(RING)]
          + [pltpu.VMEM((CHUNK, HH), jnp.float32) for _ in range(RING)]
          + [pltpu.VMEM((CHUNK,), jnp.int32) for _ in range(RING)]
          + [pltpu.VMEM((CHUNK,), jnp.int32) for _ in range(RING)]
          + [pltpu.VMEM((CHUNK,), jnp.float32) for _ in range(RING)]
          + [pltpu.VMEM((WL_CHUNK,), jnp.int32)]
          + [pltpu.SemaphoreType.DMA for _ in range(3 * RING + 1)]
      ),
      compiler_params=pltpu.CompilerParams(use_tc_tiling_on_sc=False,
                                          needs_layout_passes=False),
  )
  return f(src3, dst3, ew3, W_A, W_B, wl_indices, wl_table_pad)


BN = 2000


def _tc_body(pA, pB, x, wl_e, b_edge, c1W, c1b, nW0, nW1, nb, c2W, c2b,
             fW, fb, out):
  f32 = jnp.float32
  s1A = pA[0] + pA[1] + b_edge[0, :HH]
  s1B = pB[0] + pB[1] + b_edge[0, HH:]
  ri = lax.broadcasted_iota(jnp.int32, (HH, H), 0)
  ci = lax.broadcasted_iota(jnp.int32, (HH, H), 1)
  m1A = c1W[:HH, :] + (ci == ri).astype(f32)
  m1B = c1W[HH:, :] + (ci == ri + HH).astype(f32)
  t = (jnp.dot(s1A, m1A, preferred_element_type=f32)
       + jnp.dot(s1B, m1B, preferred_element_type=f32) + c1b[:])
  xn = (jnp.dot(x[:], nW0[:], preferred_element_type=f32)
        + jnp.dot(wl_e[:, :WL_DIM], nW1[:], preferred_element_type=f32)
        + nb[:])
  t = t + xn + jnp.dot(xn, c2W[:], preferred_element_type=f32) + c2b[:]
  out[:] = jnp.dot(jnp.maximum(t, 0.0), fW[:], preferred_element_type=f32) + fb[:]


def _tc_dense(pA, pB, x, wl_e, b_edge, c1W, c1b, nW0, nW1, nb, c2W, c2b,
              fW, fb):
  grid = (N // BN,)
  in_specs = [
      pl.BlockSpec((NC, BN, HH), lambda i: (0, i, 0)),
      pl.BlockSpec((NC, BN, HH), lambda i: (0, i, 0)),
      pl.BlockSpec((BN, IN), lambda i: (i, 0)),
      pl.BlockSpec((BN, HH), lambda i: (i, 0)),
      pl.BlockSpec((1, H), lambda i: (0, 0)),
      pl.BlockSpec((H, H), lambda i: (0, 0)),
      pl.BlockSpec((1, H), lambda i: (0, 0)),
      pl.BlockSpec((IN, H), lambda i: (0, 0)),
      pl.BlockSpec((WL_DIM, H), lambda i: (0, 0)),
      pl.BlockSpec((1, H), lambda i: (0, 0)),
      pl.BlockSpec((H, H), lambda i: (0, 0)),
      pl.BlockSpec((1, H), lambda i: (0, 0)),
      pl.BlockSpec((H, H), lambda i: (0, 0)),
      pl.BlockSpec((1, H), lambda i: (0, 0)),
  ]
  return pl.pallas_call(
      _tc_body,
      grid=grid,
      in_specs=in_specs,
      out_specs=pl.BlockSpec((BN, H), lambda i: (i, 0)),
      out_shape=jax.ShapeDtypeStruct((N, H), jnp.float32),
  )(pA, pB, x, wl_e, b_edge, c1W, c1b, nW0, nW1, nb, c2W, c2b, fW, fb)


def kernel(wl_indices, x, edge_index, edge_weight, W_edge, b_edge, wl_table,
           node_W, node_b, cat1_W, cat1_b, cat2_W, cat2_b, final_W, final_b):
  src3 = edge_index[0].reshape(NW, NCHUNK, CHUNK)
  dst3 = edge_index[1].reshape(NW, NCHUNK, CHUNK)
  ew3 = edge_weight.reshape(NW, NCHUNK, CHUNK)
  blk = jnp.stack([jnp.arange(16), jnp.arange(16) + 16], axis=1).reshape(32)
  perm = jnp.concatenate([blk + 32 * g for g in range(HH // 32)])
  W_A = W_edge[:, :HH][:, perm].astype(jnp.bfloat16)
  W_B = W_edge[:, HH:][:, perm].astype(jnp.bfloat16)
  wl_pad = jnp.pad(wl_table, ((0, 0), (0, HH - WL_DIM)))
  pA, pB, wl_e = _sc_spmm(src3, dst3, ew3, W_A, W_B, wl_indices, wl_pad)
  nW0 = node_W[:IN]
  nW1 = node_W[IN:]
  r = lambda v: v.reshape(1, -1)
  return _tc_dense(pA, pB, x, wl_e, r(b_edge), cat1_W, r(cat1_b), nW0, nW1,
                   r(node_b), cat2_W, r(cat2_b), final_W, r(final_b))
